# Initial kernel scaffold; baseline (speedup 1.0000x reference)
import functools

import jax
import jax.numpy as jnp
from jax import lax
from jax.experimental import pallas as pl
from jax.experimental.pallas import tpu as pltpu
from jax.experimental.pallas import tpu_sc as plsc

D = 64
CH = 128


@functools.lru_cache(maxsize=None)
def _make_kernel(B: int):
    info = plsc.get_sparse_core_info()
    NC, NS = info.num_cores, info.num_subcores
    NW = NC * NS
    assert B % (NW * CH) == 0
    b_per_w = B // NW
    n_ch = b_per_w // CH
    assert n_ch % 2 == 0

    mesh = plsc.VectorSubcoreMesh(core_axis_name="c", subcore_axis_name="s")

    @functools.partial(
        pl.kernel,
        mesh=mesh,
        out_type=jax.ShapeDtypeStruct((B, D), jnp.float32),
        scratch_types=[
            pltpu.VMEM((n_ch, CH), jnp.int32),
            pltpu.VMEM((CH, D), jnp.float32),
            pltpu.VMEM((CH, D), jnp.float32),
            pltpu.SemaphoreType.DMA,
            pltpu.SemaphoreType.DMA,
        ],
    )
    def k(idx_hbm, table_hbm, out_hbm, idx_v, rows0, rows1, g0, g1):
        wid = lax.axis_index("s") * NC + lax.axis_index("c")
        base = wid * b_per_w
        pltpu.sync_copy(idx_hbm.at[wid], idx_v)

        @pl.loop(0, n_ch, step=2)
        def _(j):
            c0 = pltpu.async_copy(table_hbm.at[idx_v.at[j]], rows0, g0)
            c1 = pltpu.async_copy(table_hbm.at[idx_v.at[j + 1]], rows1, g1)
            c0.wait()
            pltpu.sync_copy(rows0, out_hbm.at[pl.ds(base + j * CH, CH)])
            c1.wait()
            pltpu.sync_copy(rows1, out_hbm.at[pl.ds(base + (j + 1) * CH, CH)])

    return k


def kernel(x, table):
    B0, H = x.shape
    B = B0 * H
    k = _make_kernel(B)
    info = plsc.get_sparse_core_info()
    NW = info.num_cores * info.num_subcores
    idx = x.reshape(NW, (B // NW) // CH, CH).astype(jnp.int32)
    out = k(idx, table)
    return out.reshape(B0, H, D)


# speedup vs baseline: 4.4243x; 4.4243x over previous
import functools

import jax
import jax.numpy as jnp
from jax import lax
from jax.experimental import pallas as pl
from jax.experimental.pallas import tpu as pltpu
from jax.experimental.pallas import tpu_sc as plsc

D = 64
CH = 128


@functools.lru_cache(maxsize=None)
def _make_kernel(B: int):
    info = plsc.get_sparse_core_info()
    NC, NS = info.num_cores, info.num_subcores
    NW = NC * NS
    assert B % (NW * CH) == 0
    b_per_w = B // NW
    n_ch = b_per_w // CH
    assert n_ch % 2 == 0

    mesh = plsc.VectorSubcoreMesh(core_axis_name="c", subcore_axis_name="s")

    @functools.partial(
        pl.kernel,
        mesh=mesh,
        out_type=jax.ShapeDtypeStruct((B, D), jnp.float32),
        compiler_params=pltpu.CompilerParams(use_tc_tiling_on_sc=False),
        scratch_types=[
            pltpu.VMEM((n_ch, CH), jnp.int32),
            pltpu.VMEM((CH, D), jnp.float32),
            pltpu.VMEM((CH, D), jnp.float32),
            pltpu.SemaphoreType.DMA,
            pltpu.SemaphoreType.DMA,
        ],
    )
    def k(idx_hbm, table_hbm, out_hbm, idx_v, rows0, rows1, g0, g1):
        wid = lax.axis_index("s") * NC + lax.axis_index("c")
        base = wid * b_per_w
        pltpu.sync_copy(idx_hbm.at[wid], idx_v)

        @pl.loop(0, n_ch, step=2)
        def _(j):
            c0 = pltpu.async_copy(table_hbm.at[idx_v.at[j]], rows0, g0)
            c1 = pltpu.async_copy(table_hbm.at[idx_v.at[j + 1]], rows1, g1)
            c0.wait()
            pltpu.sync_copy(rows0, out_hbm.at[pl.ds(base + j * CH, CH)])
            c1.wait()
            pltpu.sync_copy(rows1, out_hbm.at[pl.ds(base + (j + 1) * CH, CH)])

    return k


def kernel(x, table):
    B0, H = x.shape
    B = B0 * H
    k = _make_kernel(B)
    info = plsc.get_sparse_core_info()
    NW = info.num_cores * info.num_subcores
    idx = x.reshape(NW, (B // NW) // CH, CH).astype(jnp.int32)
    out = k(idx, table)
    return out.reshape(B0, H, D)
